# Initial kernel scaffold; baseline (speedup 1.0000x reference)
import jax
import jax.numpy as jnp
from jax.experimental import pallas as pl


def kernel(x, edge_index, W1, b1, g1, be1, W2, b2, g2, be2, W3, b3, g3, be3, W4, b4):
    raise NotImplementedError("write your pallas kernel here")


# speedup vs baseline: 5.2385x; 5.2385x over previous
import functools

import jax
import jax.numpy as jnp
from jax import lax
from jax.experimental import pallas as pl
from jax.experimental.pallas import tpu as pltpu
from jax.experimental.pallas import tpu_sc as plsc

N = 10000
E = 160000
C = 256
H = 128
NC = 2
NS = 16
CHUNK = 128
EPT = 5120
E_PAD = 163840
NROWS = E_PAD // CHUNK
ROWS_PER_TILE = NROWS // NS
DEG_ROWS_PER_TILE = NROWS // (NC * NS)
ACC_ROWS = N + NS
SLAB = 624
TAIL_OFF = SLAB * NS
TAIL = N - TAIL_OFF

def _sc_agg_body(hp_hbm, src_hbm, dst_hbm, out_hbm, src_v, dst_v, rows_v, acc_sh, sem):
    c = lax.axis_index("c")
    s = lax.axis_index("s")
    pltpu.sync_copy(hp_hbm.at[pl.ds(c * N + s * SLAB, SLAB)],
                    acc_sh.at[pl.ds(s * SLAB, SLAB)])

    @pl.when(s == 0)
    def _():
        pltpu.sync_copy(hp_hbm.at[pl.ds(c * N + TAIL_OFF, TAIL)],
                        acc_sh.at[pl.ds(TAIL_OFF, TAIL)])

    pltpu.sync_copy(src_hbm.at[c, pl.ds(s * ROWS_PER_TILE, ROWS_PER_TILE)], src_v)
    pltpu.sync_copy(dst_hbm.at[pl.ds(s * ROWS_PER_TILE, ROWS_PER_TILE)], dst_v)
    plsc.subcore_barrier()

    def body(j, _):
        pltpu.async_copy(hp_hbm.at[src_v.at[j]], rows_v, sem).wait()
        pltpu.sync_copy(rows_v, acc_sh.at[dst_v.at[j]], add=True)
        return 0

    lax.fori_loop(0, ROWS_PER_TILE, body, 0)
    plsc.subcore_barrier()
    pltpu.sync_copy(acc_sh.at[pl.ds(s * SLAB, SLAB)],
                    out_hbm.at[pl.ds(c * N + s * SLAB, SLAB)])

    @pl.when(s == 0)
    def _():
        pltpu.sync_copy(acc_sh.at[pl.ds(TAIL_OFF, TAIL)],
                        out_hbm.at[pl.ds(c * N + TAIL_OFF, TAIL)])


@functools.lru_cache(maxsize=None)
def _sc_kernels():
    mesh = plsc.VectorSubcoreMesh(core_axis_name="c", subcore_axis_name="s",
                                  num_cores=NC, num_subcores=NS)
    sc_agg = pl.kernel(
        _sc_agg_body,
        out_type=jax.ShapeDtypeStruct((NC * N, H), jnp.float32),
        mesh=mesh,
        scratch_types=[
            pltpu.VMEM((ROWS_PER_TILE, CHUNK), jnp.int32),
            pltpu.VMEM((ROWS_PER_TILE, CHUNK), jnp.int32),
            pltpu.VMEM((CHUNK, H), jnp.float32),
            pltpu.VMEM_SHARED((ACC_ROWS, H), jnp.float32),
            pltpu.SemaphoreType.DMA,
        ],
    )
    return sc_agg


_HIGH = lax.Precision.HIGHEST
RB = 1000
GRID = N // RB


def _tc_h1_body(x_ref, w_ref, deg_ref, out_ref):
    dinv = lax.rsqrt(deg_ref[...])
    h = jnp.dot(x_ref[...], w_ref[...], precision=_HIGH) * dinv
    out_ref[0] = h[:, :H]
    out_ref[1] = h[:, H:]


def _tc_h1(x, w, deg):
    return pl.pallas_call(
        _tc_h1_body,
        grid=(GRID,),
        in_specs=[
            pl.BlockSpec((RB, C), lambda i: (i, 0)),
            pl.BlockSpec((C, C), lambda i: (0, 0)),
            pl.BlockSpec((RB, 1), lambda i: (i, 0)),
        ],
        out_specs=pl.BlockSpec((NC, RB, H), lambda i: (0, i, 0)),
        out_shape=jax.ShapeDtypeStruct((NC, N, H), jnp.float32),
    )(x, w, deg)


def _tc_stats_body(acc_ref, deg_ref, b_ref, out_ref):
    dinv = lax.rsqrt(deg_ref[...])
    conv = (jnp.concatenate([acc_ref[0], acc_ref[1]], axis=1)
            * dinv + b_ref[...])
    s1 = jnp.sum(conv, axis=0, keepdims=True)
    s2 = jnp.sum(conv * conv, axis=0, keepdims=True)
    out_ref[...] = jnp.concatenate(
        [s1, s2, jnp.zeros((6, C), jnp.float32)], axis=0)


def _tc_stats(acc, deg, b):
    return pl.pallas_call(
        _tc_stats_body,
        out_shape=jax.ShapeDtypeStruct((8, C), jnp.float32),
    )(acc, deg, b)


def _tc_apply_body(acc_ref, deg_ref, b_ref, st_ref, g_ref, be_ref, w_ref,
                   y_ref, hp_ref, *, res_ref=None):
    dinv = lax.rsqrt(deg_ref[...])
    conv = (jnp.concatenate([acc_ref[0], acc_ref[1]], axis=1)
            * dinv + b_ref[...])
    mu = st_ref[0:1] * (1.0 / N)
    var = st_ref[1:2] * (1.0 / N) - mu * mu
    t = g_ref[...] * (conv - mu) * lax.rsqrt(var + 1e-5) + be_ref[...]
    if res_ref is not None:
        t = t + res_ref[...]
    y = jnp.maximum(t, 0.0)
    y_ref[...] = y
    h = jnp.dot(y, w_ref[...], precision=_HIGH) * dinv
    hp_ref[0] = h[:, :H]
    hp_ref[1] = h[:, H:]


def _tc_apply(acc, deg, b, st, g, be, w, res=None):
    in_specs = [
        pl.BlockSpec((NC, RB, H), lambda i: (0, i, 0)),
        pl.BlockSpec((RB, 1), lambda i: (i, 0)),
        pl.BlockSpec((1, C), lambda i: (0, 0)),
        pl.BlockSpec((8, C), lambda i: (0, 0)),
        pl.BlockSpec((1, C), lambda i: (0, 0)),
        pl.BlockSpec((1, C), lambda i: (0, 0)),
        pl.BlockSpec((C, C), lambda i: (0, 0)),
    ]
    args = [acc, deg, b, st, g, be, w]
    if res is None:
        body = functools.partial(_tc_apply_body)
    else:
        in_specs.append(pl.BlockSpec((RB, C), lambda i: (i, 0)))
        args.append(res)

        def body(a, d, bb, s, gg, bee, ww, r, y_ref, hp_ref):
            _tc_apply_body(a, d, bb, s, gg, bee, ww, y_ref, hp_ref, res_ref=r)

    return pl.pallas_call(
        body,
        grid=(GRID,),
        in_specs=in_specs,
        out_specs=[
            pl.BlockSpec((RB, C), lambda i: (i, 0)),
            pl.BlockSpec((NC, RB, H), lambda i: (0, i, 0)),
        ],
        out_shape=[
            jax.ShapeDtypeStruct((N, C), jnp.float32),
            jax.ShapeDtypeStruct((NC, N, H), jnp.float32),
        ],
    )(*args)


def _tc_final_body(acc_ref, deg_ref, b_ref, out_ref):
    dinv = lax.rsqrt(deg_ref[...])
    out_ref[...] = (jnp.concatenate([acc_ref[0], acc_ref[1]], axis=1)
                    * dinv + b_ref[...])


def _tc_final(acc, deg, b):
    return pl.pallas_call(
        _tc_final_body,
        grid=(GRID,),
        in_specs=[
            pl.BlockSpec((NC, RB, H), lambda i: (0, i, 0)),
            pl.BlockSpec((RB, 1), lambda i: (i, 0)),
            pl.BlockSpec((1, C), lambda i: (0, 0)),
        ],
        out_specs=pl.BlockSpec((RB, C), lambda i: (i, 0)),
        out_shape=jax.ShapeDtypeStruct((N, C), jnp.float32),
    )(acc, deg, b)


def kernel(x, edge_index, W1, b1, g1, be1, W2, b2, g2, be2, W3, b3, g3, be3,
           W4, b4):
    ei = edge_index.astype(jnp.int32)
    src, dst = ei[0], ei[1]
    pad = E_PAD - E
    src_p = jnp.concatenate([src, jnp.zeros((pad,), jnp.int32)]).reshape(NROWS, CHUNK)
    src2 = jnp.stack([src_p, src_p + N])
    dst_p = jnp.concatenate([dst, jnp.full((pad,), N, jnp.int32)]).reshape(NROWS, CHUNK)

    _sc_agg = _sc_kernels()
    ones_flat = jnp.ones((NC * N, H), jnp.float32)
    deg = lax.slice(_sc_agg(ones_flat, src2, dst_p), (0, 0), (N, 1))

    b1r, b2r, b3r, b4r = (b.reshape(1, C) for b in (b1, b2, b3, b4))
    g1r, g2r, g3r = (g.reshape(1, C) for g in (g1, g2, g3))
    be1r, be2r, be3r = (b.reshape(1, C) for b in (be1, be2, be3))

    hp1 = _tc_h1(x, W1, deg)
    acc1 = _sc_agg(hp1.reshape(NC * N, H), src2, dst_p).reshape(NC, N, H)
    st1 = _tc_stats(acc1, deg, b1r)
    y1, hp2 = _tc_apply(acc1, deg, b1r, st1, g1r, be1r, W2)

    acc2 = _sc_agg(hp2.reshape(NC * N, H), src2, dst_p).reshape(NC, N, H)
    st2 = _tc_stats(acc2, deg, b2r)
    y2, hp3 = _tc_apply(acc2, deg, b2r, st2, g2r, be2r, W3, res=y1)

    acc3 = _sc_agg(hp3.reshape(NC * N, H), src2, dst_p).reshape(NC, N, H)
    st3 = _tc_stats(acc3, deg, b3r)
    _, hp4 = _tc_apply(acc3, deg, b3r, st3, g3r, be3r, W4, res=y2)

    acc4 = _sc_agg(hp4.reshape(NC * N, H), src2, dst_p).reshape(NC, N, H)
    return _tc_final(acc4, deg, b4r)


# speedup vs baseline: 7.5229x; 1.4361x over previous
import functools

import jax
import jax.numpy as jnp
from jax import lax
from jax.experimental import pallas as pl
from jax.experimental.pallas import tpu as pltpu
from jax.experimental.pallas import tpu_sc as plsc

N = 10000
E = 160000
C = 256
H = 128
NC = 2
NS = 16
CHUNK = 128
ROWS_PER_TILE = 80
NROWS = NS * ROWS_PER_TILE
E_PAD = NROWS * CHUNK
IB = 16
NB = ROWS_PER_TILE // IB
ACC_ROWS = N + 8
SLAB = 624
TAIL_OFF = SLAB * NS
TAIL = N - TAIL_OFF

def _sc_agg_body(do_gather, hp_hbm, src_hbm, dst_hbm, out_hbm,
                 src_i, dst_i, rows_a, rows_b, acc_sh,
                 sem_ga, sem_gb, sem_si, sem_di):
    c = lax.axis_index("c")
    s = lax.axis_index("s")
    pltpu.sync_copy(hp_hbm.at[pl.ds(c * N + s * SLAB, SLAB)],
                    acc_sh.at[pl.ds(s * SLAB, SLAB)])

    @pl.when(s == 0)
    def _():
        pltpu.sync_copy(hp_hbm.at[pl.ds(c * N + TAIL_OFF, TAIL)],
                        acc_sh.at[pl.ds(TAIL_OFF, TAIL)])

    base = s * ROWS_PER_TILE
    if do_gather:
        pltpu.sync_copy(src_hbm.at[c, pl.ds(base, IB)], src_i.at[0])
    pltpu.sync_copy(dst_hbm.at[pl.ds(base, IB)], dst_i.at[0])
    if not do_gather:
        pltpu.sync_copy(hp_hbm.at[pl.ds(0, CHUNK)], rows_a)
        pltpu.sync_copy(hp_hbm.at[pl.ds(0, CHUNK)], rows_b)
    plsc.subcore_barrier()

    if do_gather:
        pltpu.async_copy(hp_hbm.at[src_i.at[0, 0]], rows_a, sem_ga)

    def block(b, _):
        bb = b % 2

        @pl.when(b < NB - 1)
        def _():
            if do_gather:
                pltpu.async_copy(src_hbm.at[c, pl.ds(base + (b + 1) * IB, IB)],
                                 src_i.at[1 - bb], sem_si)
            pltpu.async_copy(dst_hbm.at[pl.ds(base + (b + 1) * IB, IB)],
                             dst_i.at[1 - bb], sem_di)

        def pair(i2, _):
            ia = 2 * i2
            if do_gather:
                pltpu.async_copy(hp_hbm.at[src_i.at[bb, ia + 1]], rows_b, sem_gb)
                pltpu.make_async_copy(hp_hbm.at[src_i.at[bb, ia]], rows_a,
                                      sem_ga).wait()
            pltpu.sync_copy(rows_a, acc_sh.at[dst_i.at[bb, ia]], add=True)
            ib_ = ia + 1
            if do_gather:
                @pl.when(i2 < IB // 2 - 1)
                def _():
                    pltpu.async_copy(hp_hbm.at[src_i.at[bb, ib_ + 1]], rows_a,
                                     sem_ga)

                @pl.when(jnp.logical_and(i2 == IB // 2 - 1, b < NB - 1))
                def _():
                    pltpu.make_async_copy(
                        src_hbm.at[c, pl.ds(base + (b + 1) * IB, IB)],
                        src_i.at[1 - bb], sem_si).wait()
                    pltpu.make_async_copy(
                        dst_hbm.at[pl.ds(base + (b + 1) * IB, IB)],
                        dst_i.at[1 - bb], sem_di).wait()
                    pltpu.async_copy(hp_hbm.at[src_i.at[1 - bb, 0]], rows_a,
                                     sem_ga)

                pltpu.make_async_copy(hp_hbm.at[src_i.at[bb, ib_]], rows_b,
                                      sem_gb).wait()
            pltpu.sync_copy(rows_b, acc_sh.at[dst_i.at[bb, ib_]], add=True)
            return 0

        lax.fori_loop(0, IB // 2, pair, 0)
        if not do_gather:
            @pl.when(b < NB - 1)
            def _():
                pltpu.make_async_copy(dst_hbm.at[pl.ds(base + (b + 1) * IB, IB)],
                                      dst_i.at[1 - bb], sem_di).wait()
        return 0

    lax.fori_loop(0, NB, block, 0)
    plsc.subcore_barrier()
    pltpu.sync_copy(acc_sh.at[pl.ds(s * SLAB, SLAB)],
                    out_hbm.at[pl.ds(c * N + s * SLAB, SLAB)])

    @pl.when(s == 0)
    def _():
        pltpu.sync_copy(acc_sh.at[pl.ds(TAIL_OFF, TAIL)],
                        out_hbm.at[pl.ds(c * N + TAIL_OFF, TAIL)])


@functools.lru_cache(maxsize=None)
def _sc_kernels():
    mesh = plsc.VectorSubcoreMesh(core_axis_name="c", subcore_axis_name="s",
                                  num_cores=NC, num_subcores=NS)
    scratch = [
        pltpu.VMEM((2, IB, CHUNK), jnp.int32),
        pltpu.VMEM((2, IB, CHUNK), jnp.int32),
        pltpu.VMEM((CHUNK, H), jnp.float32),
        pltpu.VMEM((CHUNK, H), jnp.float32),
        pltpu.VMEM_SHARED((ACC_ROWS, H), jnp.float32),
        pltpu.SemaphoreType.DMA,
        pltpu.SemaphoreType.DMA,
        pltpu.SemaphoreType.DMA,
        pltpu.SemaphoreType.DMA,
    ]
    out_type = jax.ShapeDtypeStruct((NC * N, H), jnp.float32)
    sc_agg = pl.kernel(
        functools.partial(_sc_agg_body, True),
        out_type=out_type, mesh=mesh, scratch_types=scratch,
    )
    sc_deg = pl.kernel(
        functools.partial(_sc_agg_body, False),
        out_type=out_type, mesh=mesh, scratch_types=scratch,
    )
    return sc_agg, sc_deg


_HIGH = lax.Precision.HIGHEST
RB = 1000
GRID = N // RB


def _tc_h1_body(x_ref, w_ref, deg_ref, out_ref):
    dinv = lax.rsqrt(deg_ref[...])
    h = jnp.dot(x_ref[...], w_ref[...], precision=_HIGH) * dinv
    out_ref[0] = h[:, :H]
    out_ref[1] = h[:, H:]


def _tc_h1(x, w, deg):
    return pl.pallas_call(
        _tc_h1_body,
        grid=(GRID,),
        in_specs=[
            pl.BlockSpec((RB, C), lambda i: (i, 0)),
            pl.BlockSpec((C, C), lambda i: (0, 0)),
            pl.BlockSpec((RB, 1), lambda i: (i, 0)),
        ],
        out_specs=pl.BlockSpec((NC, RB, H), lambda i: (0, i, 0)),
        out_shape=jax.ShapeDtypeStruct((NC, N, H), jnp.float32),
    )(x, w, deg)


def _tc_stats_body(acc_ref, deg_ref, b_ref, out_ref):
    dinv = lax.rsqrt(deg_ref[...])
    conv = (jnp.concatenate([acc_ref[0], acc_ref[1]], axis=1)
            * dinv + b_ref[...])
    s1 = jnp.sum(conv, axis=0, keepdims=True)
    s2 = jnp.sum(conv * conv, axis=0, keepdims=True)
    out_ref[...] = jnp.concatenate(
        [s1, s2, jnp.zeros((6, C), jnp.float32)], axis=0)


def _tc_stats(acc, deg, b):
    return pl.pallas_call(
        _tc_stats_body,
        out_shape=jax.ShapeDtypeStruct((8, C), jnp.float32),
    )(acc, deg, b)


def _tc_apply_body(acc_ref, deg_ref, b_ref, st_ref, g_ref, be_ref, w_ref,
                   y_ref, hp_ref, *, res_ref=None):
    dinv = lax.rsqrt(deg_ref[...])
    conv = (jnp.concatenate([acc_ref[0], acc_ref[1]], axis=1)
            * dinv + b_ref[...])
    mu = st_ref[0:1] * (1.0 / N)
    var = st_ref[1:2] * (1.0 / N) - mu * mu
    t = g_ref[...] * (conv - mu) * lax.rsqrt(var + 1e-5) + be_ref[...]
    if res_ref is not None:
        t = t + res_ref[...]
    y = jnp.maximum(t, 0.0)
    y_ref[...] = y
    h = jnp.dot(y, w_ref[...], precision=_HIGH) * dinv
    hp_ref[0] = h[:, :H]
    hp_ref[1] = h[:, H:]


def _tc_apply(acc, deg, b, st, g, be, w, res=None):
    in_specs = [
        pl.BlockSpec((NC, RB, H), lambda i: (0, i, 0)),
        pl.BlockSpec((RB, 1), lambda i: (i, 0)),
        pl.BlockSpec((1, C), lambda i: (0, 0)),
        pl.BlockSpec((8, C), lambda i: (0, 0)),
        pl.BlockSpec((1, C), lambda i: (0, 0)),
        pl.BlockSpec((1, C), lambda i: (0, 0)),
        pl.BlockSpec((C, C), lambda i: (0, 0)),
    ]
    args = [acc, deg, b, st, g, be, w]
    if res is None:
        body = functools.partial(_tc_apply_body)
    else:
        in_specs.append(pl.BlockSpec((RB, C), lambda i: (i, 0)))
        args.append(res)

        def body(a, d, bb, s, gg, bee, ww, r, y_ref, hp_ref):
            _tc_apply_body(a, d, bb, s, gg, bee, ww, y_ref, hp_ref, res_ref=r)

    return pl.pallas_call(
        body,
        grid=(GRID,),
        in_specs=in_specs,
        out_specs=[
            pl.BlockSpec((RB, C), lambda i: (i, 0)),
            pl.BlockSpec((NC, RB, H), lambda i: (0, i, 0)),
        ],
        out_shape=[
            jax.ShapeDtypeStruct((N, C), jnp.float32),
            jax.ShapeDtypeStruct((NC, N, H), jnp.float32),
        ],
    )(*args)


def _tc_final_body(acc_ref, deg_ref, b_ref, out_ref):
    dinv = lax.rsqrt(deg_ref[...])
    out_ref[...] = (jnp.concatenate([acc_ref[0], acc_ref[1]], axis=1)
                    * dinv + b_ref[...])


def _tc_final(acc, deg, b):
    return pl.pallas_call(
        _tc_final_body,
        grid=(GRID,),
        in_specs=[
            pl.BlockSpec((NC, RB, H), lambda i: (0, i, 0)),
            pl.BlockSpec((RB, 1), lambda i: (i, 0)),
            pl.BlockSpec((1, C), lambda i: (0, 0)),
        ],
        out_specs=pl.BlockSpec((RB, C), lambda i: (i, 0)),
        out_shape=jax.ShapeDtypeStruct((N, C), jnp.float32),
    )(acc, deg, b)


def kernel(x, edge_index, W1, b1, g1, be1, W2, b2, g2, be2, W3, b3, g3, be3,
           W4, b4):
    ei = edge_index.astype(jnp.int32)
    src, dst = ei[0], ei[1]
    pad = E_PAD - E
    src_p = jnp.concatenate([src, jnp.zeros((pad,), jnp.int32)]).reshape(NROWS, CHUNK)
    src2 = jnp.stack([src_p, src_p + N])
    dst_p = jnp.concatenate([dst, jnp.full((pad,), N, jnp.int32)]).reshape(NROWS, CHUNK)

    _sc_agg, _sc_deg = _sc_kernels()
    ones_flat = jnp.ones((NC * N, H), jnp.float32)
    deg = lax.slice(_sc_deg(ones_flat, src2, dst_p), (0, 0), (N, 1))

    b1r, b2r, b3r, b4r = (b.reshape(1, C) for b in (b1, b2, b3, b4))
    g1r, g2r, g3r = (g.reshape(1, C) for g in (g1, g2, g3))
    be1r, be2r, be3r = (b.reshape(1, C) for b in (be1, be2, be3))

    hp1 = _tc_h1(x, W1, deg)
    acc1 = _sc_agg(hp1.reshape(NC * N, H), src2, dst_p).reshape(NC, N, H)
    st1 = _tc_stats(acc1, deg, b1r)
    y1, hp2 = _tc_apply(acc1, deg, b1r, st1, g1r, be1r, W2)

    acc2 = _sc_agg(hp2.reshape(NC * N, H), src2, dst_p).reshape(NC, N, H)
    st2 = _tc_stats(acc2, deg, b2r)
    y2, hp3 = _tc_apply(acc2, deg, b2r, st2, g2r, be2r, W3, res=y1)

    acc3 = _sc_agg(hp3.reshape(NC * N, H), src2, dst_p).reshape(NC, N, H)
    st3 = _tc_stats(acc3, deg, b3r)
    _, hp4 = _tc_apply(acc3, deg, b3r, st3, g3r, be3r, W4, res=y2)

    acc4 = _sc_agg(hp4.reshape(NC * N, H), src2, dst_p).reshape(NC, N, H)
    return _tc_final(acc4, deg, b4r)
